# baseline (speedup 1.0000x reference)
import functools

import jax
import jax.numpy as jnp
from jax import lax
from jax.experimental import pallas as pl
from jax.experimental.pallas import tpu as pltpu
from jax.experimental.pallas import tpu_sc as plsc

B, L, D = 16384, 50, 32
N = B * L
NUM_CORES = 2
NUM_SUBCORES = 16
NW = NUM_CORES * NUM_SUBCORES
PER_W = N // NW
CHUNK = 1024
NCHUNK = PER_W // CHUNK


def _mesh():
    return plsc.VectorSubcoreMesh(core_axis_name="c", subcore_axis_name="s")


@functools.partial(
    pl.kernel,
    out_type=jax.ShapeDtypeStruct((N, D), jnp.float32),
    mesh=_mesh(),
    compiler_params=pltpu.CompilerParams(
        use_tc_tiling_on_sc=False, needs_layout_passes=False
    ),
    scratch_types=[
        pltpu.VMEM((CHUNK,), jnp.int32),
        pltpu.VMEM((CHUNK, D), jnp.float32),
        pltpu.SemaphoreType.DMA,
    ],
)
def _gather_kernel(table_hbm, idx_hbm, out_hbm, idx_v, rows_v, sem):
    wid = lax.axis_index("s") * NUM_CORES + lax.axis_index("c")
    base = wid * PER_W

    def chunk_body(g, carry):
        cbase = base + g * CHUNK
        pltpu.sync_copy(idx_hbm.at[pl.ds(cbase, CHUNK)], idx_v)
        pltpu.async_copy(table_hbm.at[idx_v], rows_v, sem).wait()

        def grp_body(i, carry2):
            v = idx_v[pl.ds(i * 16, 16)]
            m = v == 0
            nz = jnp.sum(jnp.where(m, 1, 0).astype(jnp.int32))

            @pl.when(nz != 0)
            def _():
                rows = i * 16 + lax.iota(jnp.int32, 16)
                zero = jnp.zeros((16,), jnp.float32)
                for c in range(D):
                    col = jnp.full((16,), c, jnp.int32)
                    plsc.store_scatter(rows_v, [rows, col], zero, mask=m)

            return carry2

        lax.fori_loop(0, CHUNK // 16, grp_body, 0)

        pltpu.sync_copy(rows_v, out_hbm.at[pl.ds(cbase, CHUNK)])
        return carry

    lax.fori_loop(0, NCHUNK, chunk_body, 0)


def kernel(table, indices):
    idx_flat = indices.reshape(N).astype(jnp.int32)
    out = _gather_kernel(table, idx_flat)
    return out.reshape(B, L, D)


# speedup vs baseline: 1.0372x; 1.0372x over previous
import functools

import jax
import jax.numpy as jnp
from jax import lax
from jax.experimental import pallas as pl
from jax.experimental.pallas import tpu as pltpu
from jax.experimental.pallas import tpu_sc as plsc

B, L, D = 16384, 50, 32
N = B * L
NUM_CORES = 2
NUM_SUBCORES = 16
NW = NUM_CORES * NUM_SUBCORES
PER_W = N // NW
CHUNK = 1024
NCHUNK = PER_W // CHUNK
NBUF = 3


def _mesh():
    return plsc.VectorSubcoreMesh(core_axis_name="c", subcore_axis_name="s")


def _fixup_zero_rows(idx_v, rows_v):
    def grp_body(i, carry):
        v = idx_v[pl.ds(i * 16, 16)]
        m = v == 0
        nz = jnp.sum(jnp.where(m, 1, 0).astype(jnp.int32))

        @pl.when(nz != 0)
        def _():
            rows = i * 16 + lax.iota(jnp.int32, 16)
            zero = jnp.zeros((16,), jnp.float32)
            for c in range(D):
                col = jnp.full((16,), c, jnp.int32)
                plsc.store_scatter(rows_v, [rows, col], zero, mask=m)

        return carry

    lax.fori_loop(0, CHUNK // 16, grp_body, 0)


@functools.partial(
    pl.kernel,
    out_type=jax.ShapeDtypeStruct((N, D), jnp.float32),
    mesh=_mesh(),
    compiler_params=pltpu.CompilerParams(
        use_tc_tiling_on_sc=False, needs_layout_passes=False
    ),
    scratch_types=(
        [pltpu.VMEM((CHUNK,), jnp.int32) for _ in range(NBUF)]
        + [pltpu.VMEM((CHUNK, D), jnp.float32) for _ in range(NBUF)]
        + [pltpu.SemaphoreType.DMA for _ in range(2 * NBUF)]
    ),
)
def _gather_kernel(table_hbm, idx_hbm, out_hbm, *bufs):
    idx_v = bufs[:NBUF]
    rows_v = bufs[NBUF : 2 * NBUF]
    gsem = bufs[2 * NBUF : 3 * NBUF]
    osem = bufs[3 * NBUF : 4 * NBUF]

    wid = lax.axis_index("s") * NUM_CORES + lax.axis_index("c")
    base = wid * PER_W

    gathers = [None] * NBUF
    writes = [None] * NBUF
    for g in range(NCHUNK + 1):
        if g < NCHUNK:
            p = g % NBUF
            cbase = base + g * CHUNK
            if writes[p] is not None:
                writes[p].wait()
                writes[p] = None
            pltpu.sync_copy(idx_hbm.at[pl.ds(cbase, CHUNK)], idx_v[p])
            gathers[p] = pltpu.async_copy(
                table_hbm.at[idx_v[p]], rows_v[p], gsem[p]
            )
        if g >= 1:
            q = (g - 1) % NBUF
            gathers[q].wait()
            _fixup_zero_rows(idx_v[q], rows_v[q])
            writes[q] = pltpu.async_copy(
                rows_v[q],
                out_hbm.at[pl.ds(base + (g - 1) * CHUNK, CHUNK)],
                osem[q],
            )
    for p in range(NBUF):
        if writes[p] is not None:
            writes[p].wait()


def kernel(table, indices):
    idx_flat = indices.reshape(N).astype(jnp.int32)
    out = _gather_kernel(table, idx_flat)
    return out.reshape(B, L, D)


# speedup vs baseline: 1.6738x; 1.6137x over previous
import functools

import jax
import jax.numpy as jnp
from jax import lax
from jax.experimental import pallas as pl
from jax.experimental.pallas import tpu as pltpu
from jax.experimental.pallas import tpu_sc as plsc

B, L, D = 16384, 50, 32
N = B * L
NUM_CORES = 2
NUM_SUBCORES = 16
NW = NUM_CORES * NUM_SUBCORES
B_PER_W = B // NW
CB = 16
CHUNK = CB * L
NCHUNK = B_PER_W // CB
NBUF = 3


def _mesh():
    return plsc.VectorSubcoreMesh(core_axis_name="c", subcore_axis_name="s")


def _fixup_zero_rows(idx_v, rows_v):
    def grp_body(i, carry):
        v = idx_v[pl.ds(i * 16, 16)]
        m = v == 0
        nz = jnp.sum(jnp.where(m, 1, 0).astype(jnp.int32))

        @pl.when(nz != 0)
        def _():
            rows = i * 16 + lax.iota(jnp.int32, 16)
            zero = jnp.zeros((16,), jnp.float32)
            for c in range(D):
                col = jnp.full((16,), c, jnp.int32)
                plsc.store_scatter(rows_v, [rows, col], zero, mask=m)

        return carry

    lax.fori_loop(0, CHUNK // 16, grp_body, 0)


@functools.partial(
    pl.kernel,
    out_type=jax.ShapeDtypeStruct((B, L, D), jnp.float32),
    mesh=_mesh(),
    compiler_params=pltpu.CompilerParams(
        use_tc_tiling_on_sc=False, needs_layout_passes=False
    ),
    scratch_types=(
        [pltpu.VMEM((CHUNK,), jnp.int32) for _ in range(NBUF)]
        + [pltpu.VMEM((CHUNK, D), jnp.float32) for _ in range(NBUF)]
        + [pltpu.SemaphoreType.DMA for _ in range(2 * NBUF)]
    ),
)
def _gather_kernel(table_hbm, idx_hbm, out_hbm, *bufs):
    idx_v = bufs[:NBUF]
    rows_v = bufs[NBUF : 2 * NBUF]
    gsem = bufs[2 * NBUF : 3 * NBUF]
    osem = bufs[3 * NBUF : 4 * NBUF]

    wid = lax.axis_index("s") * NUM_CORES + lax.axis_index("c")
    bbase = wid * B_PER_W

    gathers = [None] * NBUF
    writes = [None] * NBUF
    for g in range(NCHUNK + 1):
        if g < NCHUNK:
            p = g % NBUF
            if writes[p] is not None:
                for w in writes[p]:
                    w.wait()
                writes[p] = None
            pltpu.sync_copy(
                idx_hbm.at[pl.ds((bbase + g * CB) * L, CHUNK)], idx_v[p]
            )
            gathers[p] = pltpu.async_copy(
                table_hbm.at[idx_v[p]], rows_v[p], gsem[p]
            )
        if g >= 1:
            q = (g - 1) % NBUF
            gathers[q].wait()
            _fixup_zero_rows(idx_v[q], rows_v[q])
            wb = bbase + (g - 1) * CB
            writes[q] = [
                pltpu.async_copy(
                    rows_v[q].at[pl.ds(k * L, L)],
                    out_hbm.at[wb + k],
                    osem[q],
                )
                for k in range(CB)
            ]
    for p in range(NBUF):
        if writes[p] is not None:
            for w in writes[p]:
                w.wait()


def kernel(table, indices):
    idx_flat = indices.reshape(N).astype(jnp.int32)
    return _gather_kernel(table, idx_flat)
